# baseline (speedup 1.0000x reference)
import functools

import jax
import jax.numpy as jnp
from jax import lax
from jax.experimental import pallas as pl
from jax.experimental.pallas import tpu as pltpu
from jax.experimental.pallas import tpu_sc as plsc

NN = 10000
NE = 320000
DE = 16
H = 128
NG = 64
NL = 4

NC, NS = 2, 16
NW = NC * NS
EPW = NE // NW
CH = 80
NCHUNK = EPW // CH
NPAIR = NCHUNK // 2
RPT = 640
AGG_PAD = NS * RPT

F32 = jnp.float32

_MESH = dict(core_axis_name="c", subcore_axis_name="s",
             num_cores=NC, num_subcores=NS)


def _pack_pairs(y):
    i = lax.bitcast_convert_type(y, jnp.int32)
    r = (i + 0x8000) & jnp.int32(-65536)
    parts = []
    for j in range(H // 32):
        r1 = r[:, 32 * j:32 * j + 16]
        r2 = r[:, 32 * j + 16:32 * j + 32]
        parts.append(lax.shift_right_logical(r1, 16) | r2)
    return jnp.concatenate(parts, axis=1)


def _linear_body(x_ref, w_ref, b_ref, o_ref, *, relu, packed):
    y = jnp.dot(x_ref[...], w_ref[...], preferred_element_type=F32) + b_ref[...]
    if relu:
        y = jnp.maximum(y, 0.0)
    o_ref[...] = _pack_pairs(y) if packed else y


def _linear(x, w, b, *, relu, bm, packed=False):
    m, k = x.shape
    n = w.shape[1]
    no = n // 2 if packed else n
    return pl.pallas_call(
        functools.partial(_linear_body, relu=relu, packed=packed),
        grid=(m // bm,),
        in_specs=[
            pl.BlockSpec((bm, k), lambda i: (i, 0)),
            pl.BlockSpec((k, n), lambda i: (0, 0)),
            pl.BlockSpec((1, n), lambda i: (0, 0)),
        ],
        out_specs=pl.BlockSpec((bm, no), lambda i: (i, 0)),
        out_shape=jax.ShapeDtypeStruct((m, no), jnp.int32 if packed else F32),
    )(x, w, b.reshape(1, n))


def _ab_body(h_ref, w1_ref, w2_ref, t_ref):
    hb = h_ref[...]
    pa = _pack_pairs(jnp.dot(hb, w1_ref[...], preferred_element_type=F32))
    pb = _pack_pairs(jnp.dot(hb, w2_ref[...], preferred_element_type=F32))
    t_ref[...] = jnp.concatenate([pa, pb], axis=1)


def _ab(h, w1, w2, bm=2000):
    return pl.pallas_call(
        _ab_body,
        grid=(NN // bm,),
        in_specs=[
            pl.BlockSpec((bm, H), lambda i: (i, 0)),
            pl.BlockSpec((H, H), lambda i: (0, 0)),
            pl.BlockSpec((H, H), lambda i: (0, 0)),
        ],
        out_specs=pl.BlockSpec((bm, H), lambda i: (i, 0)),
        out_shape=jax.ShapeDtypeStruct((NN, H), jnp.int32),
    )(h, w1, w2)


def _node_body(h_ref, a0_ref, a1_ref, wn1_ref, wn2_ref, b_ref, o_ref):
    agg = a0_ref[0] + a1_ref[0]
    y = jnp.dot(h_ref[...], wn1_ref[...], preferred_element_type=F32)
    y = y + jnp.dot(agg, wn2_ref[...], preferred_element_type=F32) + b_ref[...]
    o_ref[...] = jnp.maximum(y, 0.0)


def _node(h, p, wn1, wn2, bnode, bm=2000):
    return pl.pallas_call(
        _node_body,
        grid=(NN // bm,),
        in_specs=[
            pl.BlockSpec((bm, H), lambda i: (i, 0)),
            pl.BlockSpec((1, bm, H), lambda i: (0, i, 0)),
            pl.BlockSpec((1, bm, H), lambda i: (1, i, 0)),
            pl.BlockSpec((H, H), lambda i: (0, 0)),
            pl.BlockSpec((H, H), lambda i: (0, 0)),
            pl.BlockSpec((1, H), lambda i: (0, 0)),
        ],
        out_specs=pl.BlockSpec((bm, H), lambda i: (i, 0)),
        out_shape=jax.ShapeDtypeStruct((NN, H), F32),
    )(h, p, p, wn1, wn2, bnode.reshape(1, H))


_BE_RD = 1280


def _readout_body(m_ref, src_ref, batch_ref, wpre_ref, bpre_ref,
                  wout_ref, bout_ref, o_ref, acc_ref, cnt_ref, lt_ref, le_ref):
    i = pl.program_id(0)

    @pl.when(i == 0)
    def _():
        acc_ref[...] = jnp.zeros_like(acc_ref)
        cnt_ref[...] = jnp.zeros_like(cnt_ref)
        brow = batch_ref[...]
        gcol = lax.broadcasted_iota(jnp.int32, (NG, 1), 0)
        lt_ref[...] = jnp.sum((brow < gcol).astype(jnp.int32), axis=1,
                              keepdims=True)
        le_ref[...] = jnp.sum((brow <= gcol).astype(jnp.int32), axis=1,
                              keepdims=True)

    phi = jnp.maximum(
        jnp.dot(m_ref[...], wpre_ref[...], preferred_element_type=F32)
        + bpre_ref[...], 0.0)
    srow = src_ref[0]
    onehot = ((srow >= lt_ref[...]) & (srow < le_ref[...])).astype(F32)
    acc_ref[...] += jnp.dot(onehot, phi, preferred_element_type=F32)
    cnt_ref[...] += jnp.dot(onehot, jnp.ones_like(phi), preferred_element_type=F32)

    @pl.when(i == pl.num_programs(0) - 1)
    def _():
        pooled = acc_ref[...] / jnp.maximum(cnt_ref[...], 1.0)
        o_ref[...] = (jnp.dot(pooled, wout_ref[...], preferred_element_type=F32)
                      + bout_ref[...])


def _readout(m, src3, batch2, wpre, bpre, wout, bout):
    nblk = NE // _BE_RD
    return pl.pallas_call(
        _readout_body,
        grid=(nblk,),
        in_specs=[
            pl.BlockSpec((_BE_RD, H), lambda i: (i, 0)),
            pl.BlockSpec((1, 1, _BE_RD), lambda i: (i, 0, 0)),
            pl.BlockSpec((1, NN), lambda i: (0, 0)),
            pl.BlockSpec((H, H), lambda i: (0, 0)),
            pl.BlockSpec((1, H), lambda i: (0, 0)),
            pl.BlockSpec((H, H), lambda i: (0, 0)),
            pl.BlockSpec((1, H), lambda i: (0, 0)),
        ],
        out_specs=pl.BlockSpec((NG, H), lambda i: (0, 0)),
        out_shape=jax.ShapeDtypeStruct((NG, H), F32),
        scratch_shapes=[
            pltpu.VMEM((NG, H), F32),
            pltpu.VMEM((NG, H), F32),
            pltpu.VMEM((NG, 1), jnp.int32),
            pltpu.VMEM((NG, 1), jnp.int32),
        ],
    )(m, src3, batch2, wpre, bpre.reshape(1, H), wout, bout.reshape(1, H))


def _fused_impl(src2_ref, dst2_ref, t_ref, c_ref, m_ref,
                idxs_ref, idxd_ref, ra, rb, cv, mo, semg, semw):
    c = lax.axis_index("c")
    s = lax.axis_index("s")
    wid = s * NC + c
    base = wid * EPW
    pltpu.sync_copy(src2_ref.at[wid], idxs_ref)
    pltpu.sync_copy(dst2_ref.at[wid], idxd_ref)

    def fire_g(i, slot):
        off = base + i * CH
        pltpu.async_copy(t_ref.at[idxs_ref.at[i]], ra[slot], semg[slot])
        pltpu.async_copy(t_ref.at[idxd_ref.at[i]], rb[slot], semg[slot])
        pltpu.async_copy(c_ref.at[pl.ds(off, CH)], cv[slot], semg[slot])

    def drain_g(slot):
        pltpu.make_async_copy(t_ref.at[idxs_ref.at[0]], ra[slot], semg[slot]).wait()
        pltpu.make_async_copy(t_ref.at[idxd_ref.at[0]], rb[slot], semg[slot]).wait()
        pltpu.make_async_copy(c_ref.at[pl.ds(base, CH)], cv[slot], semg[slot]).wait()

    def compute(slot):
        rab, rbb, cvb, mob = ra[slot], rb[slot], cv[slot], mo[slot]
        mask = jnp.int32(-65536)

        def unp_lo(w):
            return lax.bitcast_convert_type(w << 16, F32)

        def unp_hi(w):
            return lax.bitcast_convert_type(w & mask, F32)

        @plsc.parallel_loop(0, CH, unroll=2)
        def row(r):
            for j in range(H // 32):
                sl = pl.ds(j * 16, 16)
                wa = rab[r, sl]
                wb = rbb[r, pl.ds(64 + j * 16, 16)]
                wc = cvb[r, sl]
                lo = unp_lo(wa) + unp_lo(wb) + unp_lo(wc)
                hi = unp_hi(wa) + unp_hi(wb) + unp_hi(wc)
                mob[r, pl.ds(j * 32, 16)] = jnp.maximum(lo, 0.0)
                mob[r, pl.ds(j * 32 + 16, 16)] = jnp.maximum(hi, 0.0)

    def fire_w(i, slot):
        pltpu.async_copy(mo[slot], m_ref.at[pl.ds(base + i * CH, CH)], semw[slot])

    def drain_w(slot):
        pltpu.make_async_copy(mo[slot], m_ref.at[pl.ds(base, CH)], semw[slot]).wait()

    fire_g(0, 0)

    def body(g, carry):
        i0 = 2 * g
        i1 = i0 + 1
        fire_g(i1, 1)
        drain_g(0)

        @pl.when(g > 0)
        def _():
            drain_w(0)

        compute(0)
        fire_w(i0, 0)

        @pl.when(g < NPAIR - 1)
        def _():
            fire_g(i0 + 2, 0)

        drain_g(1)

        @pl.when(g > 0)
        def _():
            drain_w(1)

        compute(1)
        fire_w(i1, 1)
        return carry

    lax.fori_loop(0, NPAIR, body, 0)
    fire_g(NCHUNK - 1, 0)
    drain_g(0)
    drain_w(0)
    compute(0)
    fire_w(NCHUNK - 1, 0)
    drain_w(1)
    drain_w(0)


def _fused_plain_body(src2_ref, dst2_ref, t_ref, c_ref,
                      m_ref, idxs_ref, idxd_ref,
                      ra0, ra1, rb0, rb1, cv0, cv1, mo0, mo1,
                      semg0, semg1, semw0, semw1):
    _fused_impl(src2_ref, dst2_ref, t_ref, c_ref, m_ref,
                idxs_ref, idxd_ref, (ra0, ra1), (rb0, rb1), (cv0, cv1),
                (mo0, mo1), (semg0, semg1), (semw0, semw1))


@functools.lru_cache(maxsize=None)
def _sc_fused_plain_kernel():
    return pl.kernel(
        _fused_plain_body,
        out_type=jax.ShapeDtypeStruct((NE, H), F32),
        mesh=plsc.VectorSubcoreMesh(**_MESH),
        scratch_types=[
            pltpu.VMEM((NCHUNK, CH), jnp.int32),
            pltpu.VMEM((NCHUNK, CH), jnp.int32),
            pltpu.VMEM((CH, H), jnp.int32),
            pltpu.VMEM((CH, H), jnp.int32),
            pltpu.VMEM((CH, H), jnp.int32),
            pltpu.VMEM((CH, H), jnp.int32),
            pltpu.VMEM((CH, H // 2), jnp.int32),
            pltpu.VMEM((CH, H // 2), jnp.int32),
            pltpu.VMEM((CH, H), F32),
            pltpu.VMEM((CH, H), F32),
            pltpu.SemaphoreType.DMA,
            pltpu.SemaphoreType.DMA,
            pltpu.SemaphoreType.DMA,
            pltpu.SemaphoreType.DMA,
        ],
    )


def _sc_fused_plain(src2, dst2, t, cc):
    return _sc_fused_plain_kernel()(src2, dst2, t, cc)


def _scatter_body(m_ref, dst2_ref, zeros_ref, out_ref,
                  idxd_ref, mv0, mv1, agg_sh, seml0, seml1):
    c = lax.axis_index("c")
    s = lax.axis_index("s")
    wid = s * NC + c
    base = wid * EPW
    pltpu.sync_copy(dst2_ref.at[wid], idxd_ref)
    pltpu.sync_copy(zeros_ref, agg_sh.at[pl.ds(s * RPT, RPT)])
    plsc.subcore_barrier()

    mv = (mv0, mv1)
    seml = (seml0, seml1)

    def fire_l(i, slot):
        pltpu.async_copy(m_ref.at[pl.ds(base + i * CH, CH)], mv[slot], seml[slot])

    def drain_l(slot):
        pltpu.make_async_copy(m_ref.at[pl.ds(base, CH)], mv[slot], seml[slot]).wait()

    def scat(i, slot):
        pltpu.sync_copy(mv[slot], agg_sh.at[idxd_ref.at[i]], add=True)

    fire_l(0, 0)

    def body(g, carry):
        i0 = 2 * g
        i1 = i0 + 1
        fire_l(i1, 1)
        drain_l(0)
        scat(i0, 0)

        @pl.when(g < NPAIR - 1)
        def _():
            fire_l(i0 + 2, 0)

        drain_l(1)
        scat(i1, 1)
        return carry

    lax.fori_loop(0, NPAIR, body, 0)
    fire_l(NCHUNK - 1, 0)
    drain_l(0)
    scat(NCHUNK - 1, 0)
    plsc.subcore_barrier()
    pltpu.sync_copy(agg_sh.at[pl.ds(s * RPT, RPT)],
                    out_ref.at[c, pl.ds(s * RPT, RPT)])


@functools.lru_cache(maxsize=None)
def _sc_scatter_kernel():
    return pl.kernel(
        _scatter_body,
        out_type=jax.ShapeDtypeStruct((NC, AGG_PAD, H), F32),
        mesh=plsc.VectorSubcoreMesh(**_MESH),
        scratch_types=[
            pltpu.VMEM((NCHUNK, CH), jnp.int32),
            pltpu.VMEM((CH, H), F32),
            pltpu.VMEM((CH, H), F32),
            pltpu.VMEM_SHARED((AGG_PAD, H), F32),
            pltpu.SemaphoreType.DMA,
            pltpu.SemaphoreType.DMA,
        ],
    )


def _sc_scatter(m, dst2, zeros_rt):
    return _sc_scatter_kernel()(m, dst2, zeros_rt)


def kernel(x, edge_index, edge_attr, batch, enc_Wn, enc_bn, enc_We, enc_be,
           W_msg, b_msg, W_node, b_node, W_pre, b_pre, W_out, b_out):
    src = edge_index[0]
    src2 = src.reshape(NW, NCHUNK, CH)
    dst2 = edge_index[1].reshape(NW, NCHUNK, CH)
    src3 = src.reshape(NE // _BE_RD, 1, _BE_RD)
    batch2 = batch.reshape(1, NN)

    hn = _linear(x, enc_Wn, enc_bn, relu=True, bm=2000)
    e = _linear(edge_attr, enc_We, enc_be, relu=True, bm=2000)
    zeros_rt = jnp.zeros((RPT, H), F32)

    out = None
    cc = _linear(e, W_msg[0, 2 * H:], b_msg[0], relu=False, bm=2000,
                 packed=True)
    for l in range(NL):
        t = _ab(hn, W_msg[l, :H], W_msg[l, H:2 * H])
        m = _sc_fused_plain(src2, dst2, t, cc)
        if l < NL - 1:
            cc = _linear(m, W_msg[l + 1, 2 * H:], b_msg[l + 1], relu=False,
                         bm=2000, packed=True)
            p = _sc_scatter(m, dst2, zeros_rt)
            hn = _node(hn, p, W_node[l, :H], W_node[l, H:], b_node[l])
        else:
            out = _readout(m, src3, batch2, W_pre, b_pre, W_out, b_out)
    return out


# speedup vs baseline: 1.1180x; 1.1180x over previous
import functools

import jax
import jax.numpy as jnp
from jax import lax
from jax.experimental import pallas as pl
from jax.experimental.pallas import tpu as pltpu
from jax.experimental.pallas import tpu_sc as plsc

NN = 10000
NE = 320000
DE = 16
H = 128
NG = 64
NL = 4

NC, NS = 2, 16
NW = NC * NS
EPW = NE // NW
CH = 80
NCHUNK = EPW // CH
NPAIR = NCHUNK // 2
RPT = 640
AGG_PAD = NS * RPT

F32 = jnp.float32

_MESH = dict(core_axis_name="c", subcore_axis_name="s",
             num_cores=NC, num_subcores=NS)


def _pack_pairs(y):
    i = lax.bitcast_convert_type(y, jnp.int32)
    r = (i + 0x8000) & jnp.int32(-65536)
    parts = []
    for j in range(H // 32):
        r1 = r[:, 32 * j:32 * j + 16]
        r2 = r[:, 32 * j + 16:32 * j + 32]
        parts.append(lax.shift_right_logical(r1, 16) | r2)
    return jnp.concatenate(parts, axis=1)


def _linear_body(x_ref, w_ref, b_ref, o_ref, *, relu, packed):
    y = jnp.dot(x_ref[...], w_ref[...], preferred_element_type=F32) + b_ref[...]
    if relu:
        y = jnp.maximum(y, 0.0)
    o_ref[...] = _pack_pairs(y) if packed else y


def _linear(x, w, b, *, relu, bm, packed=False):
    m, k = x.shape
    n = w.shape[1]
    no = n // 2 if packed else n
    return pl.pallas_call(
        functools.partial(_linear_body, relu=relu, packed=packed),
        grid=(m // bm,),
        in_specs=[
            pl.BlockSpec((bm, k), lambda i: (i, 0)),
            pl.BlockSpec((k, n), lambda i: (0, 0)),
            pl.BlockSpec((1, n), lambda i: (0, 0)),
        ],
        out_specs=pl.BlockSpec((bm, no), lambda i: (i, 0)),
        out_shape=jax.ShapeDtypeStruct((m, no), jnp.int32 if packed else F32),
    )(x, w, b.reshape(1, n))


def _ab_body(h_ref, w1_ref, w2_ref, a_ref, b_ref):
    hb = h_ref[...]
    a_ref[...] = jnp.dot(hb, w1_ref[...], preferred_element_type=F32)
    b_ref[...] = jnp.dot(hb, w2_ref[...], preferred_element_type=F32)


def _ab(h, w1, w2, bm=2000):
    return pl.pallas_call(
        _ab_body,
        grid=(NN // bm,),
        in_specs=[
            pl.BlockSpec((bm, H), lambda i: (i, 0)),
            pl.BlockSpec((H, H), lambda i: (0, 0)),
            pl.BlockSpec((H, H), lambda i: (0, 0)),
        ],
        out_specs=[pl.BlockSpec((bm, H), lambda i: (i, 0))] * 2,
        out_shape=[jax.ShapeDtypeStruct((NN, H), F32)] * 2,
    )(h, w1, w2)


def _node_body(h_ref, a0_ref, a1_ref, wn1_ref, wn2_ref, b_ref, o_ref):
    agg = a0_ref[0] + a1_ref[0]
    y = jnp.dot(h_ref[...], wn1_ref[...], preferred_element_type=F32)
    y = y + jnp.dot(agg, wn2_ref[...], preferred_element_type=F32) + b_ref[...]
    o_ref[...] = jnp.maximum(y, 0.0)


def _node(h, p, wn1, wn2, bnode, bm=2000):
    return pl.pallas_call(
        _node_body,
        grid=(NN // bm,),
        in_specs=[
            pl.BlockSpec((bm, H), lambda i: (i, 0)),
            pl.BlockSpec((1, bm, H), lambda i: (0, i, 0)),
            pl.BlockSpec((1, bm, H), lambda i: (1, i, 0)),
            pl.BlockSpec((H, H), lambda i: (0, 0)),
            pl.BlockSpec((H, H), lambda i: (0, 0)),
            pl.BlockSpec((1, H), lambda i: (0, 0)),
        ],
        out_specs=pl.BlockSpec((bm, H), lambda i: (i, 0)),
        out_shape=jax.ShapeDtypeStruct((NN, H), F32),
    )(h, p, p, wn1, wn2, bnode.reshape(1, H))


_BE_RD = 1280


def _readout_body(m_ref, src_ref, batch_ref, wpre_ref, bpre_ref,
                  wout_ref, bout_ref, o_ref, acc_ref, cnt_ref, lt_ref, le_ref):
    i = pl.program_id(0)

    @pl.when(i == 0)
    def _():
        acc_ref[...] = jnp.zeros_like(acc_ref)
        cnt_ref[...] = jnp.zeros_like(cnt_ref)
        brow = batch_ref[...]
        gcol = lax.broadcasted_iota(jnp.int32, (NG, 1), 0)
        lt_ref[...] = jnp.sum((brow < gcol).astype(jnp.int32), axis=1,
                              keepdims=True)
        le_ref[...] = jnp.sum((brow <= gcol).astype(jnp.int32), axis=1,
                              keepdims=True)

    phi = jnp.maximum(
        jnp.dot(m_ref[...], wpre_ref[...], preferred_element_type=F32)
        + bpre_ref[...], 0.0)
    srow = src_ref[0]
    onehot = ((srow >= lt_ref[...]) & (srow < le_ref[...])).astype(F32)
    acc_ref[...] += jnp.dot(onehot, phi, preferred_element_type=F32)
    cnt_ref[...] += jnp.dot(onehot, jnp.ones_like(phi), preferred_element_type=F32)

    @pl.when(i == pl.num_programs(0) - 1)
    def _():
        pooled = acc_ref[...] / jnp.maximum(cnt_ref[...], 1.0)
        o_ref[...] = (jnp.dot(pooled, wout_ref[...], preferred_element_type=F32)
                      + bout_ref[...])


def _readout(m, src3, batch2, wpre, bpre, wout, bout):
    nblk = NE // _BE_RD
    return pl.pallas_call(
        _readout_body,
        grid=(nblk,),
        in_specs=[
            pl.BlockSpec((_BE_RD, H), lambda i: (i, 0)),
            pl.BlockSpec((1, 1, _BE_RD), lambda i: (i, 0, 0)),
            pl.BlockSpec((1, NN), lambda i: (0, 0)),
            pl.BlockSpec((H, H), lambda i: (0, 0)),
            pl.BlockSpec((1, H), lambda i: (0, 0)),
            pl.BlockSpec((H, H), lambda i: (0, 0)),
            pl.BlockSpec((1, H), lambda i: (0, 0)),
        ],
        out_specs=pl.BlockSpec((NG, H), lambda i: (0, 0)),
        out_shape=jax.ShapeDtypeStruct((NG, H), F32),
        scratch_shapes=[
            pltpu.VMEM((NG, H), F32),
            pltpu.VMEM((NG, H), F32),
            pltpu.VMEM((NG, 1), jnp.int32),
            pltpu.VMEM((NG, 1), jnp.int32),
        ],
    )(m, src3, batch2, wpre, bpre.reshape(1, H), wout, bout.reshape(1, H))


def _fused_impl(src2_ref, dst2_ref, a_ref, b_ref, c_ref, m_ref,
                idxs_ref, idxd_ref, ra, rb, cv, mo, semg, semw):
    c = lax.axis_index("c")
    s = lax.axis_index("s")
    wid = s * NC + c
    base = wid * EPW
    pltpu.sync_copy(src2_ref.at[wid], idxs_ref)
    pltpu.sync_copy(dst2_ref.at[wid], idxd_ref)

    def fire_g(i, slot):
        off = base + i * CH
        pltpu.async_copy(a_ref.at[idxs_ref.at[i]], ra[slot], semg[slot])
        pltpu.async_copy(b_ref.at[idxd_ref.at[i]], rb[slot], semg[slot])
        pltpu.async_copy(c_ref.at[pl.ds(off, CH)], cv[slot], semg[slot])

    def drain_g(slot):
        pltpu.make_async_copy(a_ref.at[idxs_ref.at[0]], ra[slot], semg[slot]).wait()
        pltpu.make_async_copy(b_ref.at[idxd_ref.at[0]], rb[slot], semg[slot]).wait()
        pltpu.make_async_copy(c_ref.at[pl.ds(base, CH)], cv[slot], semg[slot]).wait()

    def compute(slot):
        rab, rbb, cvb, mob = ra[slot], rb[slot], cv[slot], mo[slot]

        @plsc.parallel_loop(0, CH, unroll=4)
        def row(r):
            for j in range(H // 16):
                sl = pl.ds(j * 16, 16)
                mob[r, sl] = jnp.maximum(rab[r, sl] + rbb[r, sl] + cvb[r, sl],
                                         0.0)

    def fire_w(i, slot):
        pltpu.async_copy(mo[slot], m_ref.at[pl.ds(base + i * CH, CH)], semw[slot])

    def drain_w(slot):
        pltpu.make_async_copy(mo[slot], m_ref.at[pl.ds(base, CH)], semw[slot]).wait()

    fire_g(0, 0)

    def body(g, carry):
        i0 = 2 * g
        i1 = i0 + 1
        fire_g(i1, 1)
        drain_g(0)

        @pl.when(g > 0)
        def _():
            drain_w(0)

        compute(0)
        fire_w(i0, 0)

        @pl.when(g < NPAIR - 1)
        def _():
            fire_g(i0 + 2, 0)

        drain_g(1)

        @pl.when(g > 0)
        def _():
            drain_w(1)

        compute(1)
        fire_w(i1, 1)
        return carry

    lax.fori_loop(0, NPAIR, body, 0)
    fire_g(NCHUNK - 1, 0)
    drain_g(0)
    drain_w(0)
    compute(0)
    fire_w(NCHUNK - 1, 0)
    drain_w(1)
    drain_w(0)


def _fused_plain_body(src2_ref, dst2_ref, a_ref, b_ref, c_ref,
                      m_ref, idxs_ref, idxd_ref,
                      ra0, ra1, rb0, rb1, cv0, cv1, mo0, mo1,
                      semg0, semg1, semw0, semw1):
    _fused_impl(src2_ref, dst2_ref, a_ref, b_ref, c_ref, m_ref,
                idxs_ref, idxd_ref, (ra0, ra1), (rb0, rb1), (cv0, cv1),
                (mo0, mo1), (semg0, semg1), (semw0, semw1))


@functools.lru_cache(maxsize=None)
def _sc_fused_plain_kernel():
    return pl.kernel(
        _fused_plain_body,
        out_type=jax.ShapeDtypeStruct((NE, H), F32),
        mesh=plsc.VectorSubcoreMesh(**_MESH),
        scratch_types=[
            pltpu.VMEM((NCHUNK, CH), jnp.int32),
            pltpu.VMEM((NCHUNK, CH), jnp.int32),
            pltpu.VMEM((CH, H), F32),
            pltpu.VMEM((CH, H), F32),
            pltpu.VMEM((CH, H), F32),
            pltpu.VMEM((CH, H), F32),
            pltpu.VMEM((CH, H), F32),
            pltpu.VMEM((CH, H), F32),
            pltpu.VMEM((CH, H), F32),
            pltpu.VMEM((CH, H), F32),
            pltpu.SemaphoreType.DMA,
            pltpu.SemaphoreType.DMA,
            pltpu.SemaphoreType.DMA,
            pltpu.SemaphoreType.DMA,
        ],
    )


def _sc_fused_plain(src2, dst2, a, b, cc):
    return _sc_fused_plain_kernel()(src2, dst2, a, b, cc)


def _scatter_body(m_ref, dst2_ref, zeros_ref, out_ref,
                  idxd_ref, mv0, mv1, agg_sh, seml0, seml1):
    c = lax.axis_index("c")
    s = lax.axis_index("s")
    wid = s * NC + c
    base = wid * EPW
    pltpu.sync_copy(dst2_ref.at[wid], idxd_ref)
    pltpu.sync_copy(zeros_ref, agg_sh.at[pl.ds(s * RPT, RPT)])
    plsc.subcore_barrier()

    mv = (mv0, mv1)
    seml = (seml0, seml1)

    def fire_l(i, slot):
        pltpu.async_copy(m_ref.at[pl.ds(base + i * CH, CH)], mv[slot], seml[slot])

    def drain_l(slot):
        pltpu.make_async_copy(m_ref.at[pl.ds(base, CH)], mv[slot], seml[slot]).wait()

    def scat(i, slot):
        pltpu.sync_copy(mv[slot], agg_sh.at[idxd_ref.at[i]], add=True)

    fire_l(0, 0)

    def body(g, carry):
        i0 = 2 * g
        i1 = i0 + 1
        fire_l(i1, 1)
        drain_l(0)
        scat(i0, 0)

        @pl.when(g < NPAIR - 1)
        def _():
            fire_l(i0 + 2, 0)

        drain_l(1)
        scat(i1, 1)
        return carry

    lax.fori_loop(0, NPAIR, body, 0)
    fire_l(NCHUNK - 1, 0)
    drain_l(0)
    scat(NCHUNK - 1, 0)
    plsc.subcore_barrier()
    pltpu.sync_copy(agg_sh.at[pl.ds(s * RPT, RPT)],
                    out_ref.at[c, pl.ds(s * RPT, RPT)])


@functools.lru_cache(maxsize=None)
def _sc_scatter_kernel():
    return pl.kernel(
        _scatter_body,
        out_type=jax.ShapeDtypeStruct((NC, AGG_PAD, H), F32),
        mesh=plsc.VectorSubcoreMesh(**_MESH),
        scratch_types=[
            pltpu.VMEM((NCHUNK, CH), jnp.int32),
            pltpu.VMEM((CH, H), F32),
            pltpu.VMEM((CH, H), F32),
            pltpu.VMEM_SHARED((AGG_PAD, H), F32),
            pltpu.SemaphoreType.DMA,
            pltpu.SemaphoreType.DMA,
        ],
    )


def _sc_scatter(m, dst2, zeros_rt):
    return _sc_scatter_kernel()(m, dst2, zeros_rt)


def kernel(x, edge_index, edge_attr, batch, enc_Wn, enc_bn, enc_We, enc_be,
           W_msg, b_msg, W_node, b_node, W_pre, b_pre, W_out, b_out):
    src = edge_index[0]
    src2 = src.reshape(NW, NCHUNK, CH)
    dst2 = edge_index[1].reshape(NW, NCHUNK, CH)
    src3 = src.reshape(NE // _BE_RD, 1, _BE_RD)
    batch2 = batch.reshape(1, NN)

    hn = _linear(x, enc_Wn, enc_bn, relu=True, bm=2000)
    e = _linear(edge_attr, enc_We, enc_be, relu=True, bm=2000)
    zeros_rt = jnp.zeros((RPT, H), F32)

    out = None
    cc = _linear(e, W_msg[0, 2 * H:], b_msg[0], relu=False, bm=2000)
    for l in range(NL):
        a, b = _ab(hn, W_msg[l, :H], W_msg[l, H:2 * H])
        m = _sc_fused_plain(src2, dst2, a, b, cc)
        if l < NL - 1:
            cc = _linear(m, W_msg[l + 1, 2 * H:], b_msg[l + 1], relu=False,
                         bm=2000)
            p = _sc_scatter(m, dst2, zeros_rt)
            hn = _node(hn, p, W_node[l, :H], W_node[l, H:], b_node[l])
        else:
            out = _readout(m, src3, batch2, W_pre, b_pre, W_out, b_out)
    return out
